# baseline (speedup 1.0000x reference)
import functools

import jax
import jax.numpy as jnp
from jax import lax
from jax.experimental import pallas as pl
from jax.experimental.pallas import tpu as pltpu
from jax.experimental.pallas import tpu_sc as plsc

NC, NS, L = 2, 16, 16
NW = NC * NS
D = 64
SCALE = 8.0
CHUNK = 256
INFLIGHT = 12
TC_BLK = 8192


@functools.partial(jax.jit, static_argnames=("B",))
def _sc_gather(idx_flat, table, B):
    b_per_w = B // NW
    n_chunks = b_per_w // CHUNK
    mesh = plsc.VectorSubcoreMesh(core_axis_name="c", subcore_axis_name="s")

    @functools.partial(
        pl.kernel,
        out_type=jax.ShapeDtypeStruct((B, D), jnp.float32),
        mesh=mesh,
        scratch_types=[
            pltpu.VMEM((b_per_w,), jnp.int32),
            pltpu.VMEM_SHARED((NS, CHUNK, D), jnp.float32),
            pltpu.SemaphoreType.DMA,
            pltpu.SemaphoreType.DMA,
        ],
    )
    def k(idx_hbm, table_hbm, out_hbm, idx_v, shared, sem_g, sem_o):
        cid = lax.axis_index("c")
        sid = lax.axis_index("s")
        wid = sid * NC + cid
        base = wid * b_per_w
        pltpu.sync_copy(
            idx_hbm.at[pl.ds(pl.multiple_of(base, 256), b_per_w)], idx_v)

        def drain_group():
            for r in range(L):
                pltpu.make_async_copy(
                    table_hbm.at[pl.ds(0, 1)],
                    shared.at[sid, pl.ds(0, 1)],
                    sem_g,
                ).wait()

        def chunk_body(c, carry):
            def fire(gg, carry2):
                g16 = idx_v[pl.ds(c * CHUNK + gg * L, L)]
                for r in range(L):
                    pltpu.async_copy(
                        table_hbm.at[pl.ds(g16[r], 1)],
                        shared.at[sid, pl.ds(gg * L + r, 1)],
                        sem_g,
                    )

                @pl.when(gg >= INFLIGHT)
                def _():
                    drain_group()

                return carry2

            lax.fori_loop(0, CHUNK // L, fire, 0, unroll=False)
            for _ in range(INFLIGHT):
                drain_group()

            pltpu.async_copy(
                shared.at[sid],
                out_hbm.at[pl.ds(
                    pl.multiple_of(base + c * CHUNK, 256), CHUNK)],
                sem_o,
            )
            pltpu.make_async_copy(
                shared.at[sid], out_hbm.at[pl.ds(0, CHUNK)], sem_o
            ).wait()
            return carry

        lax.fori_loop(0, n_chunks, chunk_body, 0, unroll=False)

    return k(idx_flat, table)


def _tc_scale_body(idx_ref, rows_ref, o_ref):
    m = jnp.where(idx_ref[...] != 0, SCALE, 0.0).astype(jnp.float32)
    o_ref[...] = rows_ref[...] * m[:, None]


@functools.partial(jax.jit, static_argnames=("B",))
def _tc_scale(idx_flat, gathered, B):
    grid = B // TC_BLK
    return pl.pallas_call(
        _tc_scale_body,
        grid=(grid,),
        in_specs=[
            pl.BlockSpec((TC_BLK,), lambda i: (i,)),
            pl.BlockSpec((TC_BLK, D), lambda i: (i, 0)),
        ],
        out_specs=pl.BlockSpec((TC_BLK, D), lambda i: (i, 0)),
        out_shape=jax.ShapeDtypeStruct((B, D), jnp.float32),
    )(idx_flat, gathered)


def kernel(inputs, shared_weights):
    B = inputs.size
    idx_flat = inputs.reshape(B).astype(jnp.int32)
    gathered = _sc_gather(idx_flat, shared_weights, B)
    out = _tc_scale(idx_flat, gathered, B)
    return out.reshape(inputs.shape + (D,))
